# baseline (speedup 1.0000x reference)
import functools

import jax
import jax.numpy as jnp
import numpy as np
from jax.experimental import pallas as pl
from jax.experimental.pallas import tpu as pltpu

_F_L2 = 1
_F_BAND_END = 2
_F_EPI = 4
_F_P1 = 8
_F_L2_FIRST = 16


def _support_body(x_ref, w1_ref, out_ref):
    out_ref[...] = jnp.dot(x_ref[...], w1_ref[...],
                           preferred_element_type=jnp.float32)


def _build_schedule(nb, nk, br, kt):
    sched = []
    l2_started = set()

    def l2_flags(i):
        if i in l2_started:
            return _F_L2
        l2_started.add(i)
        return _F_L2 | _F_L2_FIRST

    for i in range(nb):
        for k in range(nk):
            f = _F_P1
            if (k + 1) * kt <= i * br:
                f |= l2_flags(i)
            if k == nk - 1:
                f |= _F_BAND_END
            sched.append((i, k, f))
    p2 = [(i, k) for i in range(nb) for k in range(nk)
          if not ((k + 1) * kt <= i * br)]
    for idx, (i, k) in enumerate(p2):
        f = l2_flags(i)
        if idx == len(p2) - 1 or p2[idx + 1][0] != i:
            f |= _F_EPI
        sched.append((i, k, f))
    arr = np.asarray(sched, dtype=np.int32)
    return arr[:, 0], arr[:, 1], arr[:, 2]


def _main_body(ii, kk, fl, adj_ref, u_ref, b1_ref, wc_ref, bc_ref, eps_ref,
               out_ref, s2_ref, o2_ref, p_ref, *, n, br, kt, nk, npad, nclass):
    t = pl.program_id(0)
    i = ii[t]
    k = kk[t]
    f = fl[t]

    @pl.when(t == 0)
    def _():
        s2_ref[n:npad, :] = jnp.zeros((npad - n, s2_ref.shape[1]), jnp.float32)

    a = adj_ref[...]
    tail = n - (nk - 1) * kt
    if tail != kt:
        a = jax.lax.cond(
            k == nk - 1,
            lambda v: jnp.where(
                jax.lax.broadcasted_iota(jnp.int32, v.shape, 1) < tail, v, 0.0),
            lambda v: v,
            a)

    @pl.when(f & _F_P1 != 0)
    def _():
        uk = u_ref[pl.ds(k * kt, kt), :]
        part = jnp.dot(a, uk, preferred_element_type=jnp.float32)

        @pl.when(k == 0)
        def _():
            p_ref[...] = part

        @pl.when(k != 0)
        def _():
            p_ref[...] += part

    @pl.when(f & _F_L2 != 0)
    def _():
        s2k = s2_ref[pl.ds(k * kt, kt), :]
        contrib = jnp.dot(a, s2k, preferred_element_type=jnp.float32)

        @pl.when(f & _F_L2_FIRST != 0)
        def _():
            o2_ref[pl.ds(i * br, br), :] = contrib

        @pl.when(f & _F_L2_FIRST == 0)
        def _():
            o2_ref[pl.ds(i * br, br), :] += contrib

    @pl.when(f & _F_BAND_END != 0)
    def _():
        h = jnp.maximum(p_ref[...] + b1_ref[...], 0.0)
        s2_ref[pl.ds(i * br, br), :] = jnp.dot(
            h, wc_ref[...], preferred_element_type=jnp.float32)

    @pl.when(f & _F_EPI != 0)
    def _():
        acc = o2_ref[pl.ds(i * br, br), :] + bc_ref[...]
        mean = acc[:, :nclass]
        logstd = acc[:, nclass:]
        z = eps_ref[pl.ds(i * br, br), :] * jnp.exp(logstd) + mean
        m = jnp.max(z, axis=1, keepdims=True)
        zs = z - m
        lse = jnp.log(jnp.sum(jnp.exp(zs), axis=1, keepdims=True))
        out_ref[...] = zs - lse


def kernel(x, adj, W1, b1, W11, b11, W12, b12):
    n, nfeat = x.shape
    nhid = W1.shape[1]
    nclass = W11.shape[1]
    nc2 = 2 * nclass

    br = 1000 if n % 1000 == 0 else 8
    kt = 1280
    nb = n // br
    nk = -(-n // kt)
    npad = nk * kt

    wc = jnp.concatenate([W11, W12], axis=1)
    bc = jnp.concatenate([b11, b12])[None, :]
    b1r = b1[None, :]
    eps = jax.random.normal(jax.random.key(42), (n, nclass), dtype=jnp.float32)

    bs = 400 if n % 400 == 0 else 8
    support = pl.pallas_call(
        _support_body,
        grid=(n // bs,),
        in_specs=[
            pl.BlockSpec((bs, nfeat), lambda i: (i, 0)),
            pl.BlockSpec((nfeat, nhid), lambda i: (0, 0)),
        ],
        out_specs=pl.BlockSpec((bs, nhid), lambda i: (i, 0)),
        out_shape=jax.ShapeDtypeStruct((n, nhid), jnp.float32),
    )(x, W1)
    u = jnp.pad(support, ((0, npad - n), (0, 0)))

    ii, kk, fl = _build_schedule(nb, nk, br, kt)

    grid_spec = pltpu.PrefetchScalarGridSpec(
        num_scalar_prefetch=3,
        grid=(len(ii),),
        in_specs=[
            pl.BlockSpec((br, kt), lambda t, ii, kk, fl: (ii[t], kk[t])),
            pl.BlockSpec((npad, nhid), lambda t, ii, kk, fl: (0, 0)),
            pl.BlockSpec((1, nhid), lambda t, ii, kk, fl: (0, 0)),
            pl.BlockSpec((nhid, nc2), lambda t, ii, kk, fl: (0, 0)),
            pl.BlockSpec((1, nc2), lambda t, ii, kk, fl: (0, 0)),
            pl.BlockSpec((n, nclass), lambda t, ii, kk, fl: (0, 0)),
        ],
        out_specs=pl.BlockSpec((br, nclass), lambda t, ii, kk, fl: (ii[t], 0)),
        scratch_shapes=[
            pltpu.VMEM((npad, nc2), jnp.float32),
            pltpu.VMEM((n, nc2), jnp.float32),
            pltpu.VMEM((br, nhid), jnp.float32),
        ],
    )

    out = pl.pallas_call(
        functools.partial(_main_body, n=n, br=br, kt=kt, nk=nk, npad=npad,
                          nclass=nclass),
        grid_spec=grid_spec,
        out_shape=jax.ShapeDtypeStruct((n, nclass), jnp.float32),
    )(jnp.asarray(ii), jnp.asarray(kk), jnp.asarray(fl),
      adj, u, b1r, wc, bc, eps)

    return out


# speedup vs baseline: 1.0555x; 1.0555x over previous
import functools

import jax
import jax.numpy as jnp
import numpy as np
from jax.experimental import pallas as pl
from jax.experimental.pallas import tpu as pltpu

_F_ASSIGN = 1
_F_BAND_END = 2
_F_EPI = 4
_F_PUB = 8


def _support_body(x_ref, w1_ref, out_ref):
    out_ref[...] = jnp.dot(x_ref[...], w1_ref[...],
                           preferred_element_type=jnp.float32)


def _build_schedule(nb, nk, br, kt):
    sched = []
    for i in range(nb):
        for k in range(nk):
            f = _F_ASSIGN if k == 0 else 0
            pub, cnt = 0, 0
            if k == nk - 1:
                f |= _F_BAND_END
                lo = (i * br) // kt
                hi = ((i + 1) * br) // kt if i < nb - 1 else nk
                if hi > lo:
                    f |= _F_PUB
                    pub, cnt = lo, hi - lo
            sched.append((i, k, f, pub, cnt))
    p2 = [(i, k) for i in range(nb) for k in range(nk)
          if k >= (i * br) // kt]
    for idx, (i, k) in enumerate(p2):
        f = _F_ASSIGN if (idx == 0 or p2[idx - 1][0] != i) else 0
        if idx == len(p2) - 1 or p2[idx + 1][0] != i:
            f |= _F_EPI
        sched.append((i, k, f, 0, 0))
    arr = np.asarray(sched, dtype=np.int32)
    return arr[:, 0], arr[:, 1], arr[:, 2], arr[:, 3], arr[:, 4]


def _main_body(ii, kk, fl, pp, qq, adj_ref, u2_ref, b1_ref, wc_ref, bc_ref,
               eps_ref, out_ref, pub_ref, stage_ref, o2_ref, acc_ref,
               *, n, br, kt, nk, npad, nhid, nclass):
    t = pl.program_id(0)
    i = ii[t]
    k = kk[t]
    f = fl[t]

    @pl.when(t == 0)
    def _():
        pub_ref[...] = jnp.zeros_like(pub_ref)
        stage_ref[...] = jnp.zeros_like(stage_ref)

    a = adj_ref[...]
    tail = n - (nk - 1) * kt
    if tail != kt:
        a = jax.lax.cond(
            k == nk - 1,
            lambda v: jnp.where(
                jax.lax.broadcasted_iota(jnp.int32, v.shape, 1) < tail, v, 0.0),
            lambda v: v,
            a)

    rhs = u2_ref[pl.ds(k * kt, kt), :] + pub_ref[pl.ds(k * kt, kt), :]
    r = jnp.dot(a, rhs, preferred_element_type=jnp.float32)

    @pl.when(f & _F_ASSIGN != 0)
    def _():
        acc_ref[...] = r

    @pl.when(f & _F_ASSIGN == 0)
    def _():
        acc_ref[...] += r

    @pl.when(f & _F_BAND_END != 0)
    def _():
        h = jnp.maximum(acc_ref[:, :nhid] + b1_ref[...], 0.0)
        stage_ref[pl.ds(i * br, br), :] = jnp.dot(
            h, wc_ref[...], preferred_element_type=jnp.float32)
        o2_ref[pl.ds(i * br, br), :] = acc_ref[:, nhid:]

    @pl.when(f & _F_PUB != 0)
    def _():
        lo = pp[t]

        def pub_one(j, carry):
            pk = lo + j
            pub_ref[pl.ds(pk * kt, kt), nhid:] = stage_ref[pl.ds(pk * kt, kt), :]
            return carry

        jax.lax.fori_loop(0, qq[t], pub_one, 0)

    @pl.when(f & _F_EPI != 0)
    def _():
        acc = o2_ref[pl.ds(i * br, br), :] + acc_ref[:, nhid:] + bc_ref[...]
        mean = acc[:, :nclass]
        logstd = acc[:, nclass:]
        z = eps_ref[pl.ds(i * br, br), :] * jnp.exp(logstd) + mean
        m = jnp.max(z, axis=1, keepdims=True)
        zs = z - m
        lse = jnp.log(jnp.sum(jnp.exp(zs), axis=1, keepdims=True))
        out_ref[...] = zs - lse


def kernel(x, adj, W1, b1, W11, b11, W12, b12):
    n, nfeat = x.shape
    nhid = W1.shape[1]
    nclass = W11.shape[1]
    nc2 = 2 * nclass
    nw = nhid + nc2

    br = 1000 if n % 1000 == 0 else 8
    kt = 1280
    nb = n // br
    nk = -(-n // kt)
    npad = nk * kt

    wc = jnp.concatenate([W11, W12], axis=1)
    bc = jnp.concatenate([b11, b12])[None, :]
    b1r = b1[None, :]
    eps = jax.random.normal(jax.random.key(42), (n, nclass), dtype=jnp.float32)

    bs = 400 if n % 400 == 0 else 8
    support = pl.pallas_call(
        _support_body,
        grid=(n // bs,),
        in_specs=[
            pl.BlockSpec((bs, nfeat), lambda i: (i, 0)),
            pl.BlockSpec((nfeat, nhid), lambda i: (0, 0)),
        ],
        out_specs=pl.BlockSpec((bs, nhid), lambda i: (i, 0)),
        out_shape=jax.ShapeDtypeStruct((n, nhid), jnp.float32),
    )(x, W1)
    u2 = jnp.pad(support, ((0, npad - n), (0, nw - nhid)))

    ii, kk, fl, pp, qq = _build_schedule(nb, nk, br, kt)

    grid_spec = pltpu.PrefetchScalarGridSpec(
        num_scalar_prefetch=5,
        grid=(len(ii),),
        in_specs=[
            pl.BlockSpec((br, kt), lambda t, ii, kk, fl, pp, qq: (ii[t], kk[t])),
            pl.BlockSpec((npad, nw), lambda t, *_: (0, 0)),
            pl.BlockSpec((1, nhid), lambda t, *_: (0, 0)),
            pl.BlockSpec((nhid, nc2), lambda t, *_: (0, 0)),
            pl.BlockSpec((1, nc2), lambda t, *_: (0, 0)),
            pl.BlockSpec((n, nclass), lambda t, *_: (0, 0)),
        ],
        out_specs=pl.BlockSpec((br, nclass),
                               lambda t, ii, kk, fl, pp, qq: (ii[t], 0)),
        scratch_shapes=[
            pltpu.VMEM((npad, nw), jnp.float32),
            pltpu.VMEM((npad, nc2), jnp.float32),
            pltpu.VMEM((n, nc2), jnp.float32),
            pltpu.VMEM((br, nw), jnp.float32),
        ],
    )

    out = pl.pallas_call(
        functools.partial(_main_body, n=n, br=br, kt=kt, nk=nk, npad=npad,
                          nhid=nhid, nclass=nclass),
        grid_spec=grid_spec,
        out_shape=jax.ShapeDtypeStruct((n, nclass), jnp.float32),
    )(jnp.asarray(ii), jnp.asarray(kk), jnp.asarray(fl), jnp.asarray(pp),
      jnp.asarray(qq), adj, u2, b1r, wc, bc, eps)

    return out


# speedup vs baseline: 1.4884x; 1.4101x over previous
import functools

import jax
import jax.numpy as jnp
from jax.experimental import pallas as pl
from jax.experimental.pallas import tpu as pltpu

_NBUF = 4
_BI = 200


def _support_body(x_ref, w1_ref, out_ref):
    out_ref[...] = jnp.dot(x_ref[...], w1_ref[...],
                           preferred_element_type=jnp.float32)


def _copy_in(adj_hbm, bufs, sems, step, slot, bi):
    pltpu.make_async_copy(
        adj_hbm.at[pl.ds(step * bi, bi), :], bufs.at[slot], sems.at[slot],
    ).start()


def _wait_in(adj_hbm, bufs, sems, step, slot, bi):
    pltpu.make_async_copy(
        adj_hbm.at[pl.ds(step * bi, bi), :], bufs.at[slot], sems.at[slot],
    ).wait()


def _layer1_body(adj_hbm, sup_ref, b1_ref, wc_ref, out_ref, bufs, sems, *, bi):
    i = pl.program_id(0)
    nsteps = pl.num_programs(0)

    @pl.when(i == 0)
    def _():
        for d in range(_NBUF):
            _copy_in(adj_hbm, bufs, sems, d, d, bi)

    cur = jax.lax.rem(i, _NBUF)
    _wait_in(adj_hbm, bufs, sems, i, cur, bi)
    h = jnp.dot(bufs[cur], sup_ref[...], preferred_element_type=jnp.float32)
    h = jnp.maximum(h + b1_ref[...], 0.0)
    out_ref[...] = jnp.dot(h, wc_ref[...], preferred_element_type=jnp.float32)

    @pl.when(i + _NBUF < nsteps)
    def _():
        _copy_in(adj_hbm, bufs, sems, i + _NBUF, cur, bi)


def _layer2_body(adj_hbm, s2_ref, bc_ref, eps_ref, out_ref, bufs, sems,
                 *, bi, nclass):
    i = pl.program_id(0)
    nsteps = pl.num_programs(0)

    @pl.when(i == 0)
    def _():
        for d in range(_NBUF):
            _copy_in(adj_hbm, bufs, sems, d, d, bi)

    cur = jax.lax.rem(i, _NBUF)
    _wait_in(adj_hbm, bufs, sems, i, cur, bi)
    acc = jnp.dot(bufs[cur], s2_ref[...], preferred_element_type=jnp.float32)
    acc = acc + bc_ref[...]
    mean = acc[:, :nclass]
    logstd = acc[:, nclass:]
    z = eps_ref[...] * jnp.exp(logstd) + mean
    m = jnp.max(z, axis=1, keepdims=True)
    zs = z - m
    lse = jnp.log(jnp.sum(jnp.exp(zs), axis=1, keepdims=True))
    out_ref[...] = zs - lse

    @pl.when(i + _NBUF < nsteps)
    def _():
        _copy_in(adj_hbm, bufs, sems, i + _NBUF, cur, bi)


def kernel(x, adj, W1, b1, W11, b11, W12, b12):
    n, nfeat = x.shape
    nhid = W1.shape[1]
    nclass = W11.shape[1]
    nc2 = 2 * nclass

    bi = _BI if n % _BI == 0 else 8
    grid = (n // bi,)

    wc = jnp.concatenate([W11, W12], axis=1)
    bc = jnp.concatenate([b11, b12])[None, :]
    b1r = b1[None, :]
    eps = jax.random.normal(jax.random.key(42), (n, nclass), dtype=jnp.float32)

    bs = 400 if n % 400 == 0 else 8
    support = pl.pallas_call(
        _support_body,
        grid=(n // bs,),
        in_specs=[
            pl.BlockSpec((bs, nfeat), lambda i: (i, 0)),
            pl.BlockSpec((nfeat, nhid), lambda i: (0, 0)),
        ],
        out_specs=pl.BlockSpec((bs, nhid), lambda i: (i, 0)),
        out_shape=jax.ShapeDtypeStruct((n, nhid), jnp.float32),
    )(x, W1)

    scratch = [
        pltpu.VMEM((_NBUF, bi, n), jnp.float32),
        pltpu.SemaphoreType.DMA((_NBUF,)),
    ]

    s2 = pl.pallas_call(
        functools.partial(_layer1_body, bi=bi),
        grid=grid,
        in_specs=[
            pl.BlockSpec(memory_space=pl.ANY),
            pl.BlockSpec((n, nhid), lambda i: (0, 0)),
            pl.BlockSpec((1, nhid), lambda i: (0, 0)),
            pl.BlockSpec((nhid, nc2), lambda i: (0, 0)),
        ],
        out_specs=pl.BlockSpec((bi, nc2), lambda i: (i, 0)),
        out_shape=jax.ShapeDtypeStruct((n, nc2), jnp.float32),
        scratch_shapes=scratch,
    )(adj, support, b1r, wc)

    out = pl.pallas_call(
        functools.partial(_layer2_body, bi=bi, nclass=nclass),
        grid=grid,
        in_specs=[
            pl.BlockSpec(memory_space=pl.ANY),
            pl.BlockSpec((n, nc2), lambda i: (0, 0)),
            pl.BlockSpec((1, nc2), lambda i: (0, 0)),
            pl.BlockSpec((bi, nclass), lambda i: (i, 0)),
        ],
        out_specs=pl.BlockSpec((bi, nclass), lambda i: (i, 0)),
        out_shape=jax.ShapeDtypeStruct((n, nclass), jnp.float32),
        scratch_shapes=scratch,
    )(adj, s2, bc, eps)

    return out


# speedup vs baseline: 1.6309x; 1.0957x over previous
import functools

import jax
import jax.numpy as jnp
from jax.experimental import pallas as pl
from jax.experimental.pallas import tpu as pltpu

_BI = 400


def _body(x_ref, adj_ref, w1_ref, b1_ref, wc_ref, bc_ref, eps_ref,
          out_ref, sup_ref, s2_ref, *, nb, bi, nclass):
    t = pl.program_id(0)

    @pl.when(t == 0)
    def _():
        sup_ref[...] = jnp.dot(x_ref[...], w1_ref[...],
                               preferred_element_type=jnp.float32)

    @pl.when((t >= 1) & (t <= nb))
    def _():
        h = jnp.dot(adj_ref[...], sup_ref[...],
                    preferred_element_type=jnp.float32)
        h = jnp.maximum(h + b1_ref[...], 0.0)
        s2_ref[pl.ds((t - 1) * bi, bi), :] = jnp.dot(
            h, wc_ref[...], preferred_element_type=jnp.float32)

    @pl.when(t > nb)
    def _():
        acc = jnp.dot(adj_ref[...], s2_ref[...],
                      preferred_element_type=jnp.float32)
        acc = acc + bc_ref[...]
        mean = acc[:, :nclass]
        logstd = acc[:, nclass:]
        z = eps_ref[...] * jnp.exp(logstd) + mean
        m = jnp.max(z, axis=1, keepdims=True)
        zs = z - m
        lse = jnp.log(jnp.sum(jnp.exp(zs), axis=1, keepdims=True))
        out_ref[...] = zs - lse


def kernel(x, adj, W1, b1, W11, b11, W12, b12):
    n, nfeat = x.shape
    nhid = W1.shape[1]
    nclass = W11.shape[1]
    nc2 = 2 * nclass

    bi = _BI if n % _BI == 0 else 8
    nb = n // bi

    wc = jnp.concatenate([W11, W12], axis=1)
    bc = jnp.concatenate([b11, b12])[None, :]
    b1r = b1[None, :]
    eps = jax.random.normal(jax.random.key(42), (n, nclass), dtype=jnp.float32)

    def adj_map(t):
        return (jnp.where(t == 0, 0, jnp.where(t <= nb, t - 1, t - 1 - nb)), 0)

    def row_map(t):
        return (jnp.where(t > nb, t - 1 - nb, 0), 0)

    out = pl.pallas_call(
        functools.partial(_body, nb=nb, bi=bi, nclass=nclass),
        grid=(2 * nb + 1,),
        in_specs=[
            pl.BlockSpec((n, nfeat), lambda t: (0, 0)),
            pl.BlockSpec((bi, n), adj_map),
            pl.BlockSpec((nfeat, nhid), lambda t: (0, 0)),
            pl.BlockSpec((1, nhid), lambda t: (0, 0)),
            pl.BlockSpec((nhid, nc2), lambda t: (0, 0)),
            pl.BlockSpec((1, nc2), lambda t: (0, 0)),
            pl.BlockSpec((bi, nclass), row_map),
        ],
        out_specs=pl.BlockSpec((bi, nclass), row_map),
        out_shape=jax.ShapeDtypeStruct((n, nclass), jnp.float32),
        scratch_shapes=[
            pltpu.VMEM((n, nhid), jnp.float32),
            pltpu.VMEM((n, nc2), jnp.float32),
        ],
    )(x, adj, W1, b1r, wc, bc, eps)

    return out
